# Initial kernel scaffold; baseline (speedup 1.0000x reference)
import jax
import jax.numpy as jnp
from jax.experimental import pallas as pl


def kernel(h_node, h_edge, edge_index, We1, be1, We2, be2, ge, bbe, Wn1, bn1, Wn2, bn2, gn, bbn):
    raise NotImplementedError("write your pallas kernel here")


# speedup vs baseline: 3.0370x; 3.0370x over previous
import functools

import jax
import jax.numpy as jnp
from jax import lax
from jax.experimental import pallas as pl
from jax.experimental.pallas import tpu as pltpu
from jax.experimental.pallas import tpu_sc as plsc

N_NODES = 10000
N_EDGES = 160000
D = 128

NC = 2
NS = 16
NW = NC * NS
CHUNK = 128
NCHUNKS = N_EDGES // CHUNK
ROWS_PER_TILE = 624
ROWS_TAIL = N_NODES - NS * ROWS_PER_TILE

_mesh = plsc.VectorSubcoreMesh(core_axis_name="c", subcore_axis_name="s")


def _gather_body(a_hbm, b_hbm, src_hbm, dst_hbm, o1_hbm, o2_hbm,
                 sidx, didx, rows_a, rows_b, sem_a, sem_b):
    c = lax.axis_index("c")
    s = lax.axis_index("s")
    wid = s * NC + c
    n_my = (NCHUNKS - wid + NW - 1) // NW

    def body(k, carry):
        base = (wid + k * NW) * CHUNK
        pltpu.sync_copy(src_hbm.at[pl.ds(base, CHUNK)], sidx)
        pltpu.sync_copy(dst_hbm.at[pl.ds(base, CHUNK)], didx)
        cp_a = pltpu.async_copy(a_hbm.at[sidx], rows_a, sem_a)
        cp_b = pltpu.async_copy(b_hbm.at[didx], rows_b, sem_b)
        cp_a.wait()
        cp_b.wait()
        pltpu.sync_copy(rows_a, o1_hbm.at[pl.ds(base, CHUNK)])
        pltpu.sync_copy(rows_b, o2_hbm.at[pl.ds(base, CHUNK)])
        return carry

    lax.fori_loop(0, n_my, body, 0)


_gather_call = functools.partial(
    pl.kernel,
    out_type=[jax.ShapeDtypeStruct((N_EDGES, D), jnp.float32),
              jax.ShapeDtypeStruct((N_EDGES, D), jnp.float32)],
    mesh=_mesh,
    scratch_types=[
        pltpu.VMEM((CHUNK,), jnp.int32),
        pltpu.VMEM((CHUNK,), jnp.int32),
        pltpu.VMEM((CHUNK, D), jnp.float32),
        pltpu.VMEM((CHUNK, D), jnp.float32),
        pltpu.SemaphoreType.DMA,
        pltpu.SemaphoreType.DMA,
    ],
)(_gather_body)


def _scatter_body(e_hbm, dst_hbm, zero_hbm, out_hbm, didx, rows_e, shared):
    c = lax.axis_index("c")
    s = lax.axis_index("s")
    wid = s * NC + c
    row0 = s * ROWS_PER_TILE
    tail0 = NS * ROWS_PER_TILE
    pltpu.sync_copy(zero_hbm.at[pl.ds(row0, ROWS_PER_TILE)],
                    shared.at[pl.ds(row0, ROWS_PER_TILE)])

    @pl.when(s == NS - 1)
    def _():
        pltpu.sync_copy(zero_hbm.at[pl.ds(tail0, ROWS_TAIL)],
                        shared.at[pl.ds(tail0, ROWS_TAIL)])

    plsc.subcore_barrier()

    n_my = (NCHUNKS - wid + NW - 1) // NW

    def body(k, carry):
        base = (wid + k * NW) * CHUNK
        pltpu.sync_copy(dst_hbm.at[pl.ds(base, CHUNK)], didx)
        pltpu.sync_copy(e_hbm.at[pl.ds(base, CHUNK)], rows_e)
        pltpu.sync_copy(rows_e, shared.at[didx], add=True)
        return carry

    lax.fori_loop(0, n_my, body, 0)
    plsc.subcore_barrier()
    pltpu.sync_copy(shared.at[pl.ds(row0, ROWS_PER_TILE)],
                    out_hbm.at[c, pl.ds(row0, ROWS_PER_TILE)])

    @pl.when(s == NS - 1)
    def _():
        pltpu.sync_copy(shared.at[pl.ds(tail0, ROWS_TAIL)],
                        out_hbm.at[c, pl.ds(tail0, ROWS_TAIL)])


_scatter_call = functools.partial(
    pl.kernel,
    out_type=jax.ShapeDtypeStruct((NC, N_NODES, D), jnp.float32),
    mesh=_mesh,
    scratch_types=[
        pltpu.VMEM((CHUNK,), jnp.int32),
        pltpu.VMEM((CHUNK, D), jnp.float32),
        pltpu.VMEM_SHARED((N_NODES, D), jnp.float32),
    ],
)(_scatter_body)


BE = 2000
BN = 2000


def _proj_body(hn, ws, wd, out_a, out_b):
    x = hn[...]
    out_a[...] = jnp.dot(x, ws[...], preferred_element_type=jnp.float32)
    out_b[...] = jnp.dot(x, wd[...], preferred_element_type=jnp.float32)


def _edge_body(g1, g2, he, w1, b1, w2, b2, g, bb, out):
    he_v = he[...]
    x = g1[...] + g2[...] + b1[...] + jnp.dot(
        he_v, w1[...], preferred_element_type=jnp.float32)
    h = jnp.maximum(x, 0.0)
    y = jnp.dot(h, w2[...], preferred_element_type=jnp.float32) + b2[...]
    mu = jnp.mean(y, axis=-1, keepdims=True)
    yc = y - mu
    var = jnp.mean(yc * yc, axis=-1, keepdims=True)
    out[...] = he_v + yc * lax.rsqrt(var + 1e-5) * g[...] + bb[...]


def _node_body(hn, p0, p1, w1a, w1b, b1, w2, b2, g, bb, ws, wd,
               out_h, out_a, out_b):
    hn_v = hn[...]
    agg = p0[...] + p1[...]
    x = (jnp.dot(hn_v, w1a[...], preferred_element_type=jnp.float32)
         + jnp.dot(agg, w1b[...], preferred_element_type=jnp.float32)
         + b1[...])
    h = jnp.maximum(x, 0.0)
    y = jnp.dot(h, w2[...], preferred_element_type=jnp.float32) + b2[...]
    mu = jnp.mean(y, axis=-1, keepdims=True)
    yc = y - mu
    var = jnp.mean(yc * yc, axis=-1, keepdims=True)
    hn_new = hn_v + yc * lax.rsqrt(var + 1e-5) * g[...] + bb[...]
    out_h[...] = hn_new
    out_a[...] = jnp.dot(hn_new, ws[...], preferred_element_type=jnp.float32)
    out_b[...] = jnp.dot(hn_new, wd[...], preferred_element_type=jnp.float32)


def _row_spec(bs):
    return pl.BlockSpec((bs, D), lambda i: (i, 0))


def _w_spec():
    return pl.BlockSpec((D, D), lambda i: (0, 0))


def _b_spec():
    return pl.BlockSpec((1, D), lambda i: (0, 0))


_proj_call = pl.pallas_call(
    _proj_body,
    grid=(N_NODES // BN,),
    in_specs=[_row_spec(BN), _w_spec(), _w_spec()],
    out_specs=[_row_spec(BN), _row_spec(BN)],
    out_shape=[jax.ShapeDtypeStruct((N_NODES, D), jnp.float32)] * 2,
)

_edge_call = pl.pallas_call(
    _edge_body,
    grid=(N_EDGES // BE,),
    in_specs=[_row_spec(BE), _row_spec(BE), _row_spec(BE),
              _w_spec(), _b_spec(), _w_spec(), _b_spec(),
              _b_spec(), _b_spec()],
    out_specs=_row_spec(BE),
    out_shape=jax.ShapeDtypeStruct((N_EDGES, D), jnp.float32),
)

_node_call = pl.pallas_call(
    _node_body,
    grid=(N_NODES // BN,),
    in_specs=[_row_spec(BN), _row_spec(BN), _row_spec(BN),
              _w_spec(), _w_spec(), _b_spec(), _w_spec(), _b_spec(),
              _b_spec(), _b_spec(), _w_spec(), _w_spec()],
    out_specs=[_row_spec(BN), _row_spec(BN), _row_spec(BN)],
    out_shape=[jax.ShapeDtypeStruct((N_NODES, D), jnp.float32)] * 3,
)


def kernel(h_node, h_edge, edge_index, We1, be1, We2, be2, ge, bbe,
           Wn1, bn1, Wn2, bn2, gn, bbn):
    src = edge_index[0]
    dst = edge_index[1]
    zeros = jnp.zeros((N_NODES, D), jnp.float32)
    num_convs = We1.shape[0]

    a_proj, b_proj = _proj_call(h_node, We1[0, :D], We1[0, D:2 * D])
    for i in range(num_convs):
        g1, g2 = _gather_call(a_proj, b_proj, src, dst)
        h_edge = _edge_call(
            g1, g2, h_edge, We1[i, 2 * D:], be1[i].reshape(1, D),
            We2[i], be2[i].reshape(1, D),
            ge[i].reshape(1, D), bbe[i].reshape(1, D))
        partials = _scatter_call(h_edge, dst, zeros)
        j = min(i + 1, num_convs - 1)
        h_node, a_proj, b_proj = _node_call(
            h_node, partials[0], partials[1],
            Wn1[i, :D], Wn1[i, D:], bn1[i].reshape(1, D),
            Wn2[i], bn2[i].reshape(1, D),
            gn[i].reshape(1, D), bbn[i].reshape(1, D),
            We1[j, :D], We1[j, D:2 * D])
    return h_node, h_edge


# speedup vs baseline: 3.2231x; 1.0613x over previous
import functools

import jax
import jax.numpy as jnp
from jax import lax
from jax.experimental import pallas as pl
from jax.experimental.pallas import tpu as pltpu
from jax.experimental.pallas import tpu_sc as plsc

N_NODES = 10000
N_EDGES = 160000
D = 128
EH = N_EDGES // 2

NC = 2
NS = 16
NW = NC * NS
CHUNK = 128
ROWS_PER_TILE = 624
ROWS_TAIL = N_NODES - NS * ROWS_PER_TILE

_mesh = plsc.VectorSubcoreMesh(core_axis_name="c", subcore_axis_name="s")


def _make_gather(n_edges):
    nchunks = n_edges // CHUNK

    def body(a_hbm, b_hbm, src_hbm, dst_hbm, o1_hbm, o2_hbm,
             sidx, didx, rows_a, rows_b, sem_a, sem_b):
        c = lax.axis_index("c")
        s = lax.axis_index("s")
        wid = s * NC + c
        n_my = (nchunks - wid + NW - 1) // NW

        def step(k, carry):
            base = (wid + k * NW) * CHUNK
            pltpu.sync_copy(src_hbm.at[pl.ds(base, CHUNK)], sidx)
            pltpu.sync_copy(dst_hbm.at[pl.ds(base, CHUNK)], didx)
            cp_a = pltpu.async_copy(a_hbm.at[sidx], rows_a, sem_a)
            cp_b = pltpu.async_copy(b_hbm.at[didx], rows_b, sem_b)
            cp_a.wait()
            cp_b.wait()
            pltpu.sync_copy(rows_a, o1_hbm.at[pl.ds(base, CHUNK)])
            pltpu.sync_copy(rows_b, o2_hbm.at[pl.ds(base, CHUNK)])
            return carry

        lax.fori_loop(0, n_my, step, 0)

    return functools.partial(
        pl.kernel,
        out_type=[jax.ShapeDtypeStruct((n_edges, D), jnp.float32),
                  jax.ShapeDtypeStruct((n_edges, D), jnp.float32)],
        mesh=_mesh,
        scratch_types=[
            pltpu.VMEM((CHUNK,), jnp.int32),
            pltpu.VMEM((CHUNK,), jnp.int32),
            pltpu.VMEM((CHUNK, D), jnp.float32),
            pltpu.VMEM((CHUNK, D), jnp.float32),
            pltpu.SemaphoreType.DMA,
            pltpu.SemaphoreType.DMA,
        ],
    )(body)


_gather_half = _make_gather(EH)


def _scatter_body(ea_hbm, eb_hbm, dsta_hbm, dstb_hbm, zero_hbm, out_hbm,
                  didx, rows_e, shared):
    c = lax.axis_index("c")
    s = lax.axis_index("s")
    wid = s * NC + c
    row0 = s * ROWS_PER_TILE
    tail0 = NS * ROWS_PER_TILE
    pltpu.sync_copy(zero_hbm.at[pl.ds(row0, ROWS_PER_TILE)],
                    shared.at[pl.ds(row0, ROWS_PER_TILE)])

    @pl.when(s == NS - 1)
    def _():
        pltpu.sync_copy(zero_hbm.at[pl.ds(tail0, ROWS_TAIL)],
                        shared.at[pl.ds(tail0, ROWS_TAIL)])

    plsc.subcore_barrier()

    nchunks = EH // CHUNK
    n_my = (nchunks - wid + NW - 1) // NW
    for e_hbm, dst_hbm in ((ea_hbm, dsta_hbm), (eb_hbm, dstb_hbm)):
        def step(k, carry, e_hbm=e_hbm, dst_hbm=dst_hbm):
            base = (wid + k * NW) * CHUNK
            pltpu.sync_copy(dst_hbm.at[pl.ds(base, CHUNK)], didx)
            pltpu.sync_copy(e_hbm.at[pl.ds(base, CHUNK)], rows_e)
            pltpu.sync_copy(rows_e, shared.at[didx], add=True)
            return carry

        lax.fori_loop(0, n_my, step, 0)

    plsc.subcore_barrier()
    pltpu.sync_copy(shared.at[pl.ds(row0, ROWS_PER_TILE)],
                    out_hbm.at[c, pl.ds(row0, ROWS_PER_TILE)])

    @pl.when(s == NS - 1)
    def _():
        pltpu.sync_copy(shared.at[pl.ds(tail0, ROWS_TAIL)],
                        out_hbm.at[c, pl.ds(tail0, ROWS_TAIL)])


_scatter_call = functools.partial(
    pl.kernel,
    out_type=jax.ShapeDtypeStruct((NC, N_NODES, D), jnp.float32),
    mesh=_mesh,
    scratch_types=[
        pltpu.VMEM((CHUNK,), jnp.int32),
        pltpu.VMEM((CHUNK, D), jnp.float32),
        pltpu.VMEM_SHARED((N_NODES, D), jnp.float32),
    ],
)(_scatter_body)


BE = 2000
BN = 2000


def _proj_body(hn, ws, wd, out_a, out_b):
    x = hn[...]
    out_a[...] = jnp.dot(x, ws[...], preferred_element_type=jnp.float32)
    out_b[...] = jnp.dot(x, wd[...], preferred_element_type=jnp.float32)


def _edge_body(g1, g2, he, w1, b1, w2, b2, g, bb, out):
    he_v = he[...]
    x = g1[...] + g2[...] + b1[...] + jnp.dot(
        he_v, w1[...], preferred_element_type=jnp.float32)
    h = jnp.maximum(x, 0.0)
    y = jnp.dot(h, w2[...], preferred_element_type=jnp.float32) + b2[...]
    mu = jnp.mean(y, axis=-1, keepdims=True)
    yc = y - mu
    var = jnp.mean(yc * yc, axis=-1, keepdims=True)
    out[...] = he_v + yc * lax.rsqrt(var + 1e-5) * g[...] + bb[...]


def _node_body(hn, p0, p1, w1a, w1b, b1, w2, b2, g, bb, ws, wd,
               out_h, out_a, out_b):
    hn_v = hn[...]
    agg = p0[...] + p1[...]
    x = (jnp.dot(hn_v, w1a[...], preferred_element_type=jnp.float32)
         + jnp.dot(agg, w1b[...], preferred_element_type=jnp.float32)
         + b1[...])
    h = jnp.maximum(x, 0.0)
    y = jnp.dot(h, w2[...], preferred_element_type=jnp.float32) + b2[...]
    mu = jnp.mean(y, axis=-1, keepdims=True)
    yc = y - mu
    var = jnp.mean(yc * yc, axis=-1, keepdims=True)
    hn_new = hn_v + yc * lax.rsqrt(var + 1e-5) * g[...] + bb[...]
    out_h[...] = hn_new
    out_a[...] = jnp.dot(hn_new, ws[...], preferred_element_type=jnp.float32)
    out_b[...] = jnp.dot(hn_new, wd[...], preferred_element_type=jnp.float32)


def _row_spec(bs):
    return pl.BlockSpec((bs, D), lambda i: (i, 0))


def _w_spec():
    return pl.BlockSpec((D, D), lambda i: (0, 0))


def _b_spec():
    return pl.BlockSpec((1, D), lambda i: (0, 0))


_proj_call = pl.pallas_call(
    _proj_body,
    grid=(N_NODES // BN,),
    in_specs=[_row_spec(BN), _w_spec(), _w_spec()],
    out_specs=[_row_spec(BN), _row_spec(BN)],
    out_shape=[jax.ShapeDtypeStruct((N_NODES, D), jnp.float32)] * 2,
)

_edge_call = pl.pallas_call(
    _edge_body,
    grid=(EH // BE,),
    in_specs=[_row_spec(BE), _row_spec(BE), _row_spec(BE),
              _w_spec(), _b_spec(), _w_spec(), _b_spec(),
              _b_spec(), _b_spec()],
    out_specs=_row_spec(BE),
    out_shape=jax.ShapeDtypeStruct((EH, D), jnp.float32),
)

_node_call = pl.pallas_call(
    _node_body,
    grid=(N_NODES // BN,),
    in_specs=[_row_spec(BN), _row_spec(BN), _row_spec(BN),
              _w_spec(), _w_spec(), _b_spec(), _w_spec(), _b_spec(),
              _b_spec(), _b_spec(), _w_spec(), _w_spec()],
    out_specs=[_row_spec(BN), _row_spec(BN), _row_spec(BN)],
    out_shape=[jax.ShapeDtypeStruct((N_NODES, D), jnp.float32)] * 3,
)


def kernel(h_node, h_edge, edge_index, We1, be1, We2, be2, ge, bbe,
           Wn1, bn1, Wn2, bn2, gn, bbn):
    src_a, src_b = edge_index[0, :EH], edge_index[0, EH:]
    dst_a, dst_b = edge_index[1, :EH], edge_index[1, EH:]
    he_a, he_b = h_edge[:EH], h_edge[EH:]
    zeros = jnp.zeros((N_NODES, D), jnp.float32)
    num_convs = We1.shape[0]

    a_proj, b_proj = _proj_call(h_node, We1[0, :D], We1[0, D:2 * D])
    for i in range(num_convs):
        ew = (We1[i, 2 * D:], be1[i].reshape(1, D), We2[i],
              be2[i].reshape(1, D), ge[i].reshape(1, D), bbe[i].reshape(1, D))
        g1a, g2a = _gather_half(a_proj, b_proj, src_a, dst_a)
        g1b, g2b = _gather_half(a_proj, b_proj, src_b, dst_b)
        he_a = _edge_call(g1a, g2a, he_a, *ew)
        he_b = _edge_call(g1b, g2b, he_b, *ew)
        partials = _scatter_call(he_a, he_b, dst_a, dst_b, zeros)
        j = min(i + 1, num_convs - 1)
        h_node, a_proj, b_proj = _node_call(
            h_node, partials[0], partials[1],
            Wn1[i, :D], Wn1[i, D:], bn1[i].reshape(1, D),
            Wn2[i], bn2[i].reshape(1, D),
            gn[i].reshape(1, D), bbn[i].reshape(1, D),
            We1[j, :D], We1[j, D:2 * D])
    return h_node, jnp.concatenate([he_a, he_b], axis=0)
